# baseline (speedup 1.0000x reference)
import functools

import jax
import jax.numpy as jnp
from jax import lax
from jax.experimental import pallas as pl
from jax.experimental.pallas import tpu as pltpu
from jax.experimental.pallas import tpu_sc as plsc

_NC = 2
_NS = 16
_NW = _NC * _NS
_LANES = 16


def _positional_encoding(static_len: int, dims: int) -> jnp.ndarray:
    pos = jnp.arange(static_len, dtype=jnp.float32)[:, None]
    i = jnp.arange(dims, dtype=jnp.float32)[None, :]
    angle = pos / jnp.power(10000.0, 2.0 * i / dims)
    even = jnp.sin(angle)
    odd = jnp.cos(angle)
    col = jnp.arange(dims)[None, :]
    pe = jnp.where(col % 2 == 0, even, odd)
    pe = pe.at[0].set(0.0)
    return pe


@functools.partial(jax.jit, static_argnames=("n_rows", "dims", "chunk", "n_chunks"))
def _sc_embed(table2, idx2, off2, pe_flat, *, n_rows, dims, chunk, n_chunks):
    rows_per_w = n_rows // _NW
    half = chunk // 2
    pe_rows = 4 * 50
    mesh = plsc.VectorSubcoreMesh(
        core_axis_name="c", subcore_axis_name="s", num_cores=_NC, num_subcores=_NS
    )
    nbuf = 3

    @functools.partial(
        pl.kernel,
        out_type=jax.ShapeDtypeStruct((n_rows // 2, 2 * dims), jnp.float32),
        mesh=mesh,
        scratch_types=[
            pltpu.VMEM((rows_per_w,), jnp.int32),
            pltpu.VMEM((rows_per_w + _LANES,), jnp.int32),
            pltpu.VMEM((pe_rows * dims,), jnp.float32),
            [pltpu.VMEM((chunk, 2 * dims), jnp.float32) for _ in range(nbuf)],
            [pltpu.SemaphoreType.DMA for _ in range(nbuf)],
            [pltpu.SemaphoreType.DMA for _ in range(nbuf)],
        ],
    )
    def body(
        table_hbm, idx_hbm, off_hbm, pe_hbm, out_hbm,
        idx_v, off_v, pe_v, rows, gsem, osem,
    ):
        wid = lax.axis_index("s") * _NC + lax.axis_index("c")
        base = wid * rows_per_w
        obase = wid * (rows_per_w // 2)
        pltpu.sync_copy(idx_hbm.at[pl.ds(base, rows_per_w)], idx_v)
        pltpu.sync_copy(
            off_hbm.at[pl.ds(base, rows_per_w)], off_v.at[pl.ds(0, rows_per_w)]
        )
        pltpu.sync_copy(pe_hbm, pe_v)

        def gather_desc(c, b):
            return pltpu.make_async_copy(
                table_hbm.at[idx_v.at[pl.ds(c * chunk, chunk)]], rows[b], gsem[b]
            )

        def out_desc(c, b):
            return pltpu.make_async_copy(
                rows[b].at[pl.ds(0, half)],
                out_hbm.at[pl.ds(obase + c * half, half)],
                osem[b],
            )

        def compute(c, b):
            rows_v = rows[b]
            phase = lax.rem(c, 5) * 10

            @pl.loop(0, half, unroll=2)
            def _pair_loop(k, rows_v=rows_v, phase=phase, c=c):
                offv = off_v[pl.ds(c * chunk + 2 * k, _LANES)]
                pe_base = (phase + 2 * k) * dims
                for t in range(2):
                    off = offv[t]
                    for j in range(dims // _LANES):
                        src = rows_v[2 * k + t, pl.ds(off + j * _LANES, _LANES)]
                        pv = pe_v[pl.ds(pe_base + t * dims + j * _LANES, _LANES)]
                        rows_v[k, pl.ds(t * dims + j * _LANES, _LANES)] = src + pv

        def step(c, b, drain, prefetch):
            gather_desc(c, b).wait()
            compute(c, b)
            out_desc(c, b).start()
            if prefetch:
                pb = (b + nbuf - 1) % nbuf
                if drain:
                    out_desc(c - 1, pb).wait()
                gather_desc(c + nbuf - 1, pb).start()

        for c in range(nbuf - 1):
            gather_desc(c, c % nbuf).start()

        for b in range(nbuf):
            step(b, b, drain=(b >= 1), prefetch=True)

        n_blocks = n_chunks // nbuf
        last_full = n_blocks - 1

        @pl.loop(1, last_full)
        def _block_loop(g):
            for b in range(nbuf):
                step(g * nbuf + b, b, drain=True, prefetch=True)

        for c in range(last_full * nbuf, n_chunks):
            step(c, c % nbuf, drain=True, prefetch=(c + nbuf - 1 < n_chunks))

        for c in range(n_chunks - nbuf, n_chunks):
            out_desc(c, c % nbuf).wait()

    return body(table2, idx2, off2, pe_flat)


def kernel(x, cutoff_max_sen_len, vocab_size, table):
    batch, seq = x.shape
    _, dims = table.shape
    n_rows = batch * seq

    chunk = 160
    assert n_rows % (_NW * chunk) == 0
    n_chunks = n_rows // (_NW * chunk)

    pe = _positional_encoding(seq, dims)
    pe_flat = jnp.tile(pe, (4, 1)).reshape(-1)

    table2 = table.reshape(-1, 2 * dims)
    flat = x.reshape(-1)
    idx2 = flat >> 1
    off2 = (flat & 1) * dims

    out2 = _sc_embed(
        table2, idx2, off2, pe_flat,
        n_rows=n_rows, dims=dims, chunk=chunk, n_chunks=n_chunks,
    )
    return out2.reshape(batch, seq, dims)
